# Initial kernel scaffold; baseline (speedup 1.0000x reference)
import jax
import jax.numpy as jnp
from jax.experimental import pallas as pl


def kernel(x, pe):
    raise NotImplementedError("write your pallas kernel here")


# speedup vs baseline: 1.4097x; 1.4097x over previous
import functools
import jax
import jax.numpy as jnp
from jax import lax
from jax.experimental import pallas as pl
from jax.experimental.pallas import tpu as pltpu, tpu_sc as plsc

MODEL_DIM = 1024
MAX_LEN = 32768
BATCH = 16384

_info = plsc.get_sparse_core_info()
_NC, _NS = _info.num_cores, _info.num_subcores
_NW = _NC * _NS
_BPW = BATCH // _NW
_CHUNK = 32
_NCHUNK = _BPW // _CHUNK


@functools.partial(
    pl.kernel,
    mesh=plsc.VectorSubcoreMesh(core_axis_name="c", subcore_axis_name="s"),
    out_type=jax.ShapeDtypeStruct((BATCH, MODEL_DIM), jnp.float32),
    scratch_types=[
        pltpu.VMEM((_BPW,), jnp.int32),
        pltpu.VMEM((_CHUNK, MODEL_DIM), jnp.float32),
        pltpu.SemaphoreType.DMA,
    ],
)
def _sc_gather(x_hbm, pe_hbm, out_hbm, idx_v, rows_v, sem):
    wid = lax.axis_index("s") * _NC + lax.axis_index("c")
    base = wid * _BPW
    pltpu.sync_copy(x_hbm.at[pl.ds(base, _BPW)], idx_v)
    for c in range(_NCHUNK):
        off = c * _CHUNK
        pltpu.async_copy(
            pe_hbm.at[idx_v.at[pl.ds(off, _CHUNK)]], rows_v, sem
        ).wait()
        pltpu.sync_copy(rows_v, out_hbm.at[pl.ds(base + off, _CHUNK)])


def kernel(x, pe):
    return _sc_gather(x.astype(jnp.int32), pe)


# speedup vs baseline: 1.6284x; 1.1552x over previous
import functools
import jax
import jax.numpy as jnp
from jax import lax
from jax.experimental import pallas as pl
from jax.experimental.pallas import tpu as pltpu, tpu_sc as plsc

MODEL_DIM = 1024
MAX_LEN = 32768
BATCH = 16384

_info = plsc.get_sparse_core_info()
_NC, _NS = _info.num_cores, _info.num_subcores
_NW = _NC * _NS
_BPW = BATCH // _NW
_CHUNK = 32
_NCHUNK = _BPW // _CHUNK


@functools.partial(
    pl.kernel,
    mesh=plsc.VectorSubcoreMesh(core_axis_name="c", subcore_axis_name="s"),
    out_type=jax.ShapeDtypeStruct((BATCH, MODEL_DIM), jnp.float32),
    scratch_types=[
        pltpu.VMEM((_BPW,), jnp.int32),
        pltpu.VMEM((_CHUNK, MODEL_DIM), jnp.float32),
        pltpu.VMEM((_CHUNK, MODEL_DIM), jnp.float32),
        pltpu.SemaphoreType.DMA,
        pltpu.SemaphoreType.DMA,
        pltpu.SemaphoreType.DMA,
        pltpu.SemaphoreType.DMA,
    ],
)
def _sc_gather(x_hbm, pe_hbm, out_hbm, idx_v, rows0, rows1, in0, in1,
               out0, out1):
    wid = lax.axis_index("s") * _NC + lax.axis_index("c")
    base = wid * _BPW
    pltpu.sync_copy(x_hbm.at[pl.ds(base, _BPW)], idx_v)

    bufs = (rows0, rows1)
    in_sems = (in0, in1)
    out_sems = (out0, out1)

    def gather(c, buf, sem):
        return pltpu.async_copy(
            pe_hbm.at[idx_v.at[pl.ds(c * _CHUNK, _CHUNK)]], buf, sem
        )

    def put(c, buf, sem):
        return pltpu.async_copy(
            buf, out_hbm.at[pl.ds(base + c * _CHUNK, _CHUNK)], sem
        )

    gathers = [None, None]
    puts = [None, None]
    gathers[0] = gather(0, bufs[0], in_sems[0])
    for c in range(_NCHUNK):
        cur = c & 1
        nxt = 1 - cur
        if c + 1 < _NCHUNK:
            if puts[nxt] is not None:
                puts[nxt].wait()
            gathers[nxt] = gather(c + 1, bufs[nxt], in_sems[nxt])
        gathers[cur].wait()
        puts[cur] = put(c, bufs[cur], out_sems[cur])
    puts[0].wait()
    puts[1].wait()


def kernel(x, pe):
    return _sc_gather(x.astype(jnp.int32), pe)


# speedup vs baseline: 1.6438x; 1.0095x over previous
import functools
import jax
import jax.numpy as jnp
from jax import lax
from jax.experimental import pallas as pl
from jax.experimental.pallas import tpu as pltpu, tpu_sc as plsc

MODEL_DIM = 1024
MAX_LEN = 32768
BATCH = 16384

_info = plsc.get_sparse_core_info()
_NC, _NS = _info.num_cores, _info.num_subcores
_NW = _NC * _NS
_BPW = BATCH // _NW
_CHUNK = 32
_NCHUNK = _BPW // _CHUNK
_NBUF = 3


@functools.partial(
    pl.kernel,
    mesh=plsc.VectorSubcoreMesh(core_axis_name="c", subcore_axis_name="s"),
    out_type=jax.ShapeDtypeStruct((BATCH, MODEL_DIM), jnp.float32),
    scratch_types=(
        [pltpu.VMEM((_BPW,), jnp.int32)]
        + [pltpu.VMEM((_CHUNK, MODEL_DIM), jnp.float32)] * _NBUF
        + [pltpu.SemaphoreType.DMA] * (2 * _NBUF)
    ),
)
def _sc_gather(x_hbm, pe_hbm, out_hbm, idx_v, *bufs_and_sems):
    bufs = bufs_and_sems[:_NBUF]
    in_sems = bufs_and_sems[_NBUF:2 * _NBUF]
    out_sems = bufs_and_sems[2 * _NBUF:]

    wid = lax.axis_index("s") * _NC + lax.axis_index("c")
    base = wid * _BPW
    pltpu.sync_copy(x_hbm.at[pl.ds(base, _BPW)], idx_v)

    def gather(c, slot):
        return pltpu.async_copy(
            pe_hbm.at[idx_v.at[pl.ds(c * _CHUNK, _CHUNK)]],
            bufs[slot], in_sems[slot],
        )

    def put(c, slot):
        return pltpu.async_copy(
            bufs[slot], out_hbm.at[pl.ds(base + c * _CHUNK, _CHUNK)],
            out_sems[slot],
        )

    gathers = [None] * _NBUF
    puts = [None] * _NBUF
    for b in range(_NBUF - 1):
        gathers[b] = gather(b, b)
    for c in range(_NCHUNK):
        slot = c % _NBUF
        pre = c + _NBUF - 1
        if pre < _NCHUNK:
            s2 = pre % _NBUF
            if puts[s2] is not None:
                puts[s2].wait()
            gathers[s2] = gather(pre, s2)
        gathers[slot].wait()
        puts[slot] = put(c, slot)
    for b in range(_NBUF):
        if puts[b] is not None:
            puts[b].wait()


def kernel(x, pe):
    return _sc_gather(x.astype(jnp.int32), pe)


# speedup vs baseline: 1.6693x; 1.0155x over previous
import functools
import jax
import jax.numpy as jnp
from jax import lax
from jax.experimental import pallas as pl
from jax.experimental.pallas import tpu as pltpu, tpu_sc as plsc

MODEL_DIM = 1024
MAX_LEN = 32768
BATCH = 16384

_info = plsc.get_sparse_core_info()
_NC, _NS = _info.num_cores, _info.num_subcores
_NW = _NC * _NS
_BPW = BATCH // _NW
_CHUNK = 16
_NCHUNK = _BPW // _CHUNK
_NBUF = 6


@functools.partial(
    pl.kernel,
    mesh=plsc.VectorSubcoreMesh(core_axis_name="c", subcore_axis_name="s"),
    out_type=jax.ShapeDtypeStruct((BATCH, MODEL_DIM), jnp.float32),
    scratch_types=(
        [pltpu.VMEM((_BPW,), jnp.int32)]
        + [pltpu.VMEM((_CHUNK, MODEL_DIM), jnp.float32)] * _NBUF
        + [pltpu.SemaphoreType.DMA] * (2 * _NBUF)
    ),
)
def _sc_gather(x_hbm, pe_hbm, out_hbm, idx_v, *bufs_and_sems):
    bufs = bufs_and_sems[:_NBUF]
    in_sems = bufs_and_sems[_NBUF:2 * _NBUF]
    out_sems = bufs_and_sems[2 * _NBUF:]

    wid = lax.axis_index("s") * _NC + lax.axis_index("c")
    base = wid * _BPW
    pltpu.sync_copy(x_hbm.at[pl.ds(base, _BPW)], idx_v)

    def gather(c, slot):
        return pltpu.async_copy(
            pe_hbm.at[idx_v.at[pl.ds(c * _CHUNK, _CHUNK)]],
            bufs[slot], in_sems[slot],
        )

    def put(c, slot):
        return pltpu.async_copy(
            bufs[slot], out_hbm.at[pl.ds(base + c * _CHUNK, _CHUNK)],
            out_sems[slot],
        )

    gathers = [None] * _NBUF
    puts = [None] * _NBUF
    for b in range(_NBUF - 1):
        gathers[b] = gather(b, b)
    for c in range(_NCHUNK):
        slot = c % _NBUF
        pre = c + _NBUF - 1
        if pre < _NCHUNK:
            s2 = pre % _NBUF
            if puts[s2] is not None:
                puts[s2].wait()
            gathers[s2] = gather(pre, s2)
        gathers[slot].wait()
        puts[slot] = put(c, slot)
    for b in range(_NBUF):
        if puts[b] is not None:
            puts[b].wait()


def kernel(x, pe):
    return _sc_gather(x.astype(jnp.int32), pe)
